# baseline (speedup 1.0000x reference)
import functools

import jax
import jax.numpy as jnp
from jax import lax
from jax.experimental import pallas as pl
from jax.experimental.pallas import tpu as pltpu
from jax.experimental.pallas import tpu_sc as plsc

N_ROWS = 320000
D = 128
T = 8
L = 16
G = D // L
NC = 2
NS = 16
NW = NC * NS
R = 400

K_SPLIT = 13
N_SC = K_SPLIT * NW * R
N_TC = N_ROWS - N_SC
ROWS_W = N_SC // NW
NCHUNK = ROWS_W // R

BT = 2560


def _sc_accumulate(y2d, types):
    mesh = plsc.VectorSubcoreMesh(
        core_axis_name="c", subcore_axis_name="s", num_cores=NC, num_subcores=NS
    )

    @functools.partial(
        pl.kernel,
        mesh=mesh,
        out_type=[
            jax.ShapeDtypeStruct((NW, T * G, L), jnp.float32),
            jax.ShapeDtypeStruct((NW, T * G, L), jnp.float32),
            jax.ShapeDtypeStruct((NW, L), jnp.float32),
        ],
        scratch_types=[
            pltpu.VMEM((R * G, L), jnp.float32),
            pltpu.VMEM((R * G, L), jnp.float32),
            pltpu.VMEM((ROWS_W,), jnp.int32),
            pltpu.VMEM((T * G, L), jnp.float32),
            pltpu.VMEM((T * G, L), jnp.float32),
            pltpu.VMEM((L,), jnp.float32),
            pltpu.SemaphoreType.DMA,
            pltpu.SemaphoreType.DMA,
        ],
        compiler_params=pltpu.CompilerParams(use_tc_tiling_on_sc=False),
    )
    def k(y_hbm, t_hbm, out_s, out_q, out_n, buf0, buf1, tv, accs, accq, accn,
          sem0, sem1):
        wid = lax.axis_index("s") * NC + lax.axis_index("c")
        base = wid * ROWS_W

        def start_y(c, buf, sem):
            pltpu.make_async_copy(
                y_hbm.at[pl.ds((base + c * R) * G, R * G)], buf, sem
            ).start()

        def wait_y(buf, sem):
            pltpu.make_async_copy(
                y_hbm.at[pl.ds(0, R * G)], buf, sem
            ).wait()

        start_y(0, buf0, sem0)
        pltpu.sync_copy(t_hbm.at[pl.ds(base, ROWS_W)], tv)

        zero = jnp.zeros((L,), jnp.float32)
        for i in range(T * G):
            accs[i, :] = zero
            accq[i, :] = zero
        accn[...] = zero
        lanes = lax.iota(jnp.int32, L)

        def compute(c, buf):
            t0 = tv[pl.ds(c * R, L)][0]
            t1 = tv[pl.ds(c * R + R - L, L)][L - 1]

            def fast():
                def body(r, acc):
                    s = list(acc[:G])
                    q = list(acc[G:])
                    rg = r * G
                    for g in range(G):
                        yv = buf[rg + g, :]
                        s[g] = s[g] + yv
                        q[g] = q[g] + yv * yv
                    return tuple(s) + tuple(q)

                res = lax.fori_loop(0, R, body, (zero,) * (2 * G), unroll=4)
                tg = t0 * G
                for g in range(G):
                    plsc.addupdate(accs.at[tg + g], res[g])
                    plsc.addupdate(accq.at[tg + g], res[G + g])
                accn[...] = accn[...] + jnp.where(
                    lanes == t0, jnp.float32(R), jnp.float32(0.0)
                )

            def slow():
                def body(j, _):
                    tvec = tv[pl.ds(c * R + j * L, L)]
                    tg0 = tvec[0]
                    tg1 = tvec[L - 1]

                    def grp_uniform():
                        def rbody(r, acc):
                            s = list(acc[:G])
                            q = list(acc[G:])
                            rg = r * G
                            for g in range(G):
                                yv = buf[rg + g, :]
                                s[g] = s[g] + yv
                                q[g] = q[g] + yv * yv
                            return tuple(s) + tuple(q)

                        res = lax.fori_loop(
                            j * L, (j + 1) * L, rbody, (zero,) * (2 * G)
                        )
                        tg = tg0 * G
                        for g in range(G):
                            plsc.addupdate(accs.at[tg + g], res[g])
                            plsc.addupdate(accq.at[tg + g], res[G + g])
                        accn[...] = accn[...] + jnp.where(
                            lanes == tg0, jnp.float32(L), jnp.float32(0.0)
                        )

                    def grp_scatter():
                        cnt = jnp.zeros((L,), jnp.float32)
                        for lane in range(L):
                            t = tvec[lane]
                            tg = t * G
                            rg = (j * L + lane) * G
                            for g in range(G):
                                yv = buf[rg + g, :]
                                plsc.addupdate(accs.at[tg + g], yv)
                                plsc.addupdate(accq.at[tg + g], yv * yv)
                            cnt = cnt + jnp.where(
                                lanes == t, jnp.float32(1.0), jnp.float32(0.0)
                            )
                        accn[...] = accn[...] + cnt

                    lax.cond(tg0 == tg1, grp_uniform, grp_scatter)
                    return 0

                lax.fori_loop(0, R // L, body, 0)

            lax.cond(t0 == t1, fast, slow)

        def pair_body(i, carry):
            c0 = 2 * i
            c1 = c0 + 1

            @pl.when(c1 < NCHUNK)
            def _():
                start_y(c1, buf1, sem1)

            wait_y(buf0, sem0)
            compute(c0, buf0)

            @pl.when(c0 + 2 < NCHUNK)
            def _():
                start_y(c0 + 2, buf0, sem0)

            @pl.when(c1 < NCHUNK)
            def _():
                wait_y(buf1, sem1)
                compute(c1, buf1)

            return carry

        lax.fori_loop(0, (NCHUNK + 1) // 2, pair_body, 0)

        pltpu.make_async_copy(accs, out_s.at[wid], sem0).start()
        pltpu.make_async_copy(accq, out_q.at[wid], sem0).start()
        pltpu.make_async_copy(accn, out_n.at[wid], sem0).start()
        pltpu.make_async_copy(accs, out_s.at[wid], sem0).wait()
        pltpu.make_async_copy(accq, out_q.at[wid], sem0).wait()
        pltpu.make_async_copy(accn, out_n.at[wid], sem0).wait()

    return k(y2d, types)


def _tc_accumulate(y, types_mat):
    nblk = N_TC // BT
    blk0 = N_SC // BT

    def body(y_ref, t_ref, os_ref, oq_ref, on_ref):
        yb = y_ref[...]
        tb = t_ref[0]
        oht = (
            jnp.broadcast_to(tb, (L, BT))
            == lax.broadcasted_iota(jnp.int32, (L, BT), 0)
        ).astype(jnp.float32)
        dn = (((1,), (0,)), ((), ()))
        s = lax.dot_general(oht, yb, dn, preferred_element_type=jnp.float32)
        q = lax.dot_general(oht, yb * yb, dn, preferred_element_type=jnp.float32)
        n = jnp.sum(oht, axis=1, keepdims=True)

        @pl.when(pl.program_id(0) == 0)
        def _():
            os_ref[...] = jnp.zeros_like(os_ref)
            oq_ref[...] = jnp.zeros_like(oq_ref)
            on_ref[...] = jnp.zeros_like(on_ref)

        os_ref[...] += s
        oq_ref[...] += q
        on_ref[...] += n

    return pl.pallas_call(
        body,
        grid=(nblk,),
        in_specs=[
            pl.BlockSpec((BT, D), lambda i: (blk0 + i, 0)),
            pl.BlockSpec((1, 1, BT), lambda i: (blk0 + i, 0, 0)),
        ],
        out_specs=[
            pl.BlockSpec((L, D), lambda i: (0, 0)),
            pl.BlockSpec((L, D), lambda i: (0, 0)),
            pl.BlockSpec((L, 1), lambda i: (0, 0)),
        ],
        out_shape=[
            jax.ShapeDtypeStruct((L, D), jnp.float32),
            jax.ShapeDtypeStruct((L, D), jnp.float32),
            jax.ShapeDtypeStruct((L, 1), jnp.float32),
        ],
    )(y, types_mat)


def _tc_fit(ps, pq, pn, ts, tq, tn):
    def body(ps_ref, pq_ref, pn_ref, ts_ref, tq_ref, tn_ref, out_ref):
        s = ps_ref[...]
        q = pq_ref[...]
        yk = ts_ref[...][:T, :]
        y2k = tq_ref[...][:T, :]
        for w in range(NW):
            yk = yk + s[w * T:(w + 1) * T, :]
            y2k = y2k + q[w * T:(w + 1) * T, :]
        nk16 = jnp.sum(pn_ref[...], axis=0, keepdims=True)
        nkb = jnp.broadcast_to(nk16, (T, L))
        row = lax.broadcasted_iota(jnp.int32, (T, L), 0)
        col = lax.broadcasted_iota(jnp.int32, (T, L), 1)
        nk = (
            jnp.sum(jnp.where(row == col, nkb, 0.0), axis=1, keepdims=True)
            + tn_ref[...][:T, :]
        )

        y_norm = jnp.sqrt(jnp.sum(yk * yk, axis=1, keepdims=True))
        y2_norm = jnp.sqrt(jnp.sum(y2k * y2k, axis=1, keepdims=True))
        nsafe = jnp.maximum(nk, 1.0)
        var = y2_norm / nsafe - (y_norm / nsafe) ** 2
        sc = jnp.sqrt(jnp.maximum(var, 1e-20))
        sc = jnp.where(nk > 0, sc, jnp.ones_like(sc))
        out_ref[...] = jnp.broadcast_to(sc, (T, D))

    out = pl.pallas_call(
        body,
        out_shape=jax.ShapeDtypeStruct((T, D), jnp.float32),
    )(ps, pq, pn, ts, tq, tn)
    return out[:, :1]


def kernel(Y, types):
    yflat = Y.reshape(N_ROWS * G, L)
    ps, pq, pn = _sc_accumulate(yflat, types)
    ts, tq, tn = _tc_accumulate(Y, types.reshape(N_ROWS // BT, 1, BT))
    return _tc_fit(ps.reshape(NW * T, D), pq.reshape(NW * T, D), pn, ts, tq, tn)


# speedup vs baseline: 1.0907x; 1.0907x over previous
import functools

import jax
import jax.numpy as jnp
from jax import lax
from jax.experimental import pallas as pl
from jax.experimental.pallas import tpu as pltpu
from jax.experimental.pallas import tpu_sc as plsc

N_ROWS = 320000
D = 128
T = 8
L = 16
G = D // L
NC = 2
NS = 16
NW = NC * NS
R = 400

K_SPLIT = 15
N_SC = K_SPLIT * NW * R
N_TC = N_ROWS - N_SC
ROWS_W = N_SC // NW
NCHUNK = ROWS_W // R

BT = 2560


def _sc_accumulate(y2d, types):
    mesh = plsc.VectorSubcoreMesh(
        core_axis_name="c", subcore_axis_name="s", num_cores=NC, num_subcores=NS
    )

    @functools.partial(
        pl.kernel,
        mesh=mesh,
        out_type=[
            jax.ShapeDtypeStruct((NW, T * G, L), jnp.float32),
            jax.ShapeDtypeStruct((NW, T * G, L), jnp.float32),
            jax.ShapeDtypeStruct((NW, L), jnp.float32),
        ],
        scratch_types=[
            pltpu.VMEM((R * G, L), jnp.float32),
            pltpu.VMEM((R * G, L), jnp.float32),
            pltpu.VMEM((ROWS_W,), jnp.int32),
            pltpu.VMEM((T * G, L), jnp.float32),
            pltpu.VMEM((T * G, L), jnp.float32),
            pltpu.VMEM((L,), jnp.float32),
            pltpu.SemaphoreType.DMA,
            pltpu.SemaphoreType.DMA,
        ],
        compiler_params=pltpu.CompilerParams(use_tc_tiling_on_sc=False),
    )
    def k(y_hbm, t_hbm, out_s, out_q, out_n, buf0, buf1, tv, accs, accq, accn,
          sem0, sem1):
        wid = lax.axis_index("s") * NC + lax.axis_index("c")
        base = wid * ROWS_W

        def start_y(c, buf, sem):
            pltpu.make_async_copy(
                y_hbm.at[pl.ds((base + c * R) * G, R * G)], buf, sem
            ).start()

        def wait_y(buf, sem):
            pltpu.make_async_copy(
                y_hbm.at[pl.ds(0, R * G)], buf, sem
            ).wait()

        start_y(0, buf0, sem0)
        pltpu.sync_copy(t_hbm.at[pl.ds(base, ROWS_W)], tv)

        zero = jnp.zeros((L,), jnp.float32)
        for i in range(T * G):
            accs[i, :] = zero
            accq[i, :] = zero
        accn[...] = zero
        lanes = lax.iota(jnp.int32, L)

        def compute(c, buf):
            t0 = tv[pl.ds(c * R, L)][0]
            t1 = tv[pl.ds(c * R + R - L, L)][L - 1]

            def fast():
                def body(r, acc):
                    s = list(acc[:G])
                    q = list(acc[G:])
                    rg = r * G
                    for g in range(G):
                        yv = buf[rg + g, :]
                        s[g] = s[g] + yv
                        q[g] = q[g] + yv * yv
                    return tuple(s) + tuple(q)

                res = lax.fori_loop(0, R, body, (zero,) * (2 * G), unroll=4)
                tg = t0 * G
                for g in range(G):
                    plsc.addupdate(accs.at[tg + g], res[g])
                    plsc.addupdate(accq.at[tg + g], res[G + g])
                accn[...] = accn[...] + jnp.where(
                    lanes == t0, jnp.float32(R), jnp.float32(0.0)
                )

            def slow():
                def body(j, _):
                    tvec = tv[pl.ds(c * R + j * L, L)]
                    tg0 = tvec[0]
                    tg1 = tvec[L - 1]

                    def grp_uniform():
                        def rbody(r, acc):
                            s = list(acc[:G])
                            q = list(acc[G:])
                            rg = r * G
                            for g in range(G):
                                yv = buf[rg + g, :]
                                s[g] = s[g] + yv
                                q[g] = q[g] + yv * yv
                            return tuple(s) + tuple(q)

                        res = lax.fori_loop(
                            j * L, (j + 1) * L, rbody, (zero,) * (2 * G)
                        )
                        tg = tg0 * G
                        for g in range(G):
                            plsc.addupdate(accs.at[tg + g], res[g])
                            plsc.addupdate(accq.at[tg + g], res[G + g])
                        accn[...] = accn[...] + jnp.where(
                            lanes == tg0, jnp.float32(L), jnp.float32(0.0)
                        )

                    def grp_scatter():
                        cnt = jnp.zeros((L,), jnp.float32)
                        for lane in range(L):
                            t = tvec[lane]
                            tg = t * G
                            rg = (j * L + lane) * G
                            for g in range(G):
                                yv = buf[rg + g, :]
                                plsc.addupdate(accs.at[tg + g], yv)
                                plsc.addupdate(accq.at[tg + g], yv * yv)
                            cnt = cnt + jnp.where(
                                lanes == t, jnp.float32(1.0), jnp.float32(0.0)
                            )
                        accn[...] = accn[...] + cnt

                    lax.cond(tg0 == tg1, grp_uniform, grp_scatter)
                    return 0

                lax.fori_loop(0, R // L, body, 0)

            lax.cond(t0 == t1, fast, slow)

        def pair_body(i, carry):
            c0 = 2 * i
            c1 = c0 + 1

            @pl.when(c1 < NCHUNK)
            def _():
                start_y(c1, buf1, sem1)

            wait_y(buf0, sem0)
            compute(c0, buf0)

            @pl.when(c0 + 2 < NCHUNK)
            def _():
                start_y(c0 + 2, buf0, sem0)

            @pl.when(c1 < NCHUNK)
            def _():
                wait_y(buf1, sem1)
                compute(c1, buf1)

            return carry

        lax.fori_loop(0, (NCHUNK + 1) // 2, pair_body, 0)

        pltpu.make_async_copy(accs, out_s.at[wid], sem0).start()
        pltpu.make_async_copy(accq, out_q.at[wid], sem0).start()
        pltpu.make_async_copy(accn, out_n.at[wid], sem0).start()
        pltpu.make_async_copy(accs, out_s.at[wid], sem0).wait()
        pltpu.make_async_copy(accq, out_q.at[wid], sem0).wait()
        pltpu.make_async_copy(accn, out_n.at[wid], sem0).wait()

    return k(y2d, types)


def _tc_accumulate(y, types_mat):
    nblk = N_TC // BT
    blk0 = N_SC // BT

    def body(y_ref, t_ref, os_ref, oq_ref, on_ref):
        yb = y_ref[...]
        tb = t_ref[0]
        oht = (
            jnp.broadcast_to(tb, (L, BT))
            == lax.broadcasted_iota(jnp.int32, (L, BT), 0)
        ).astype(jnp.float32)
        dn = (((1,), (0,)), ((), ()))
        s = lax.dot_general(oht, yb, dn, preferred_element_type=jnp.float32)
        q = lax.dot_general(oht, yb * yb, dn, preferred_element_type=jnp.float32)
        n = jnp.sum(oht, axis=1, keepdims=True)

        @pl.when(pl.program_id(0) == 0)
        def _():
            os_ref[...] = jnp.zeros_like(os_ref)
            oq_ref[...] = jnp.zeros_like(oq_ref)
            on_ref[...] = jnp.zeros_like(on_ref)

        os_ref[...] += s
        oq_ref[...] += q
        on_ref[...] += n

    return pl.pallas_call(
        body,
        grid=(nblk,),
        in_specs=[
            pl.BlockSpec((BT, D), lambda i: (blk0 + i, 0)),
            pl.BlockSpec((1, 1, BT), lambda i: (blk0 + i, 0, 0)),
        ],
        out_specs=[
            pl.BlockSpec((L, D), lambda i: (0, 0)),
            pl.BlockSpec((L, D), lambda i: (0, 0)),
            pl.BlockSpec((L, 1), lambda i: (0, 0)),
        ],
        out_shape=[
            jax.ShapeDtypeStruct((L, D), jnp.float32),
            jax.ShapeDtypeStruct((L, D), jnp.float32),
            jax.ShapeDtypeStruct((L, 1), jnp.float32),
        ],
    )(y, types_mat)


def _tc_fit(ps, pq, pn, ts, tq, tn):
    def body(ps_ref, pq_ref, pn_ref, ts_ref, tq_ref, tn_ref, out_ref):
        s = ps_ref[...]
        q = pq_ref[...]
        yk = ts_ref[...][:T, :]
        y2k = tq_ref[...][:T, :]
        for w in range(NW):
            yk = yk + s[w * T:(w + 1) * T, :]
            y2k = y2k + q[w * T:(w + 1) * T, :]
        nk16 = jnp.sum(pn_ref[...], axis=0, keepdims=True)
        nkb = jnp.broadcast_to(nk16, (T, L))
        row = lax.broadcasted_iota(jnp.int32, (T, L), 0)
        col = lax.broadcasted_iota(jnp.int32, (T, L), 1)
        nk = (
            jnp.sum(jnp.where(row == col, nkb, 0.0), axis=1, keepdims=True)
            + tn_ref[...][:T, :]
        )

        y_norm = jnp.sqrt(jnp.sum(yk * yk, axis=1, keepdims=True))
        y2_norm = jnp.sqrt(jnp.sum(y2k * y2k, axis=1, keepdims=True))
        nsafe = jnp.maximum(nk, 1.0)
        var = y2_norm / nsafe - (y_norm / nsafe) ** 2
        sc = jnp.sqrt(jnp.maximum(var, 1e-20))
        sc = jnp.where(nk > 0, sc, jnp.ones_like(sc))
        out_ref[...] = jnp.broadcast_to(sc, (T, D))

    out = pl.pallas_call(
        body,
        out_shape=jax.ShapeDtypeStruct((T, D), jnp.float32),
    )(ps, pq, pn, ts, tq, tn)
    return out[:, :1]


def kernel(Y, types):
    yflat = Y.reshape(N_ROWS * G, L)
    ps, pq, pn = _sc_accumulate(yflat, types)
    ts, tq, tn = _tc_accumulate(Y, types.reshape(N_ROWS // BT, 1, BT))
    return _tc_fit(ps.reshape(NW * T, D), pq.reshape(NW * T, D), pn, ts, tq, tn)


# speedup vs baseline: 1.0990x; 1.0076x over previous
import functools

import jax
import jax.numpy as jnp
from jax import lax
from jax.experimental import pallas as pl
from jax.experimental.pallas import tpu as pltpu
from jax.experimental.pallas import tpu_sc as plsc

N_ROWS = 320000
D = 128
T = 8
L = 16
G = D // L
NC = 2
NS = 16
NW = NC * NS
R = 400

K_SPLIT = 16
N_SC = K_SPLIT * NW * R
N_TC = N_ROWS - N_SC
ROWS_W = N_SC // NW
NCHUNK = ROWS_W // R

BT = 2560


def _sc_accumulate(y2d, types):
    mesh = plsc.VectorSubcoreMesh(
        core_axis_name="c", subcore_axis_name="s", num_cores=NC, num_subcores=NS
    )

    @functools.partial(
        pl.kernel,
        mesh=mesh,
        out_type=[
            jax.ShapeDtypeStruct((NW, T * G, L), jnp.float32),
            jax.ShapeDtypeStruct((NW, T * G, L), jnp.float32),
            jax.ShapeDtypeStruct((NW, L), jnp.float32),
        ],
        scratch_types=[
            pltpu.VMEM((R * G, L), jnp.float32),
            pltpu.VMEM((R * G, L), jnp.float32),
            pltpu.VMEM((ROWS_W,), jnp.int32),
            pltpu.VMEM((T * G, L), jnp.float32),
            pltpu.VMEM((T * G, L), jnp.float32),
            pltpu.VMEM((L,), jnp.float32),
            pltpu.SemaphoreType.DMA,
            pltpu.SemaphoreType.DMA,
        ],
        compiler_params=pltpu.CompilerParams(use_tc_tiling_on_sc=False),
    )
    def k(y_hbm, t_hbm, out_s, out_q, out_n, buf0, buf1, tv, accs, accq, accn,
          sem0, sem1):
        wid = lax.axis_index("s") * NC + lax.axis_index("c")
        base = wid * ROWS_W

        def start_y(c, buf, sem):
            pltpu.make_async_copy(
                y_hbm.at[pl.ds((base + c * R) * G, R * G)], buf, sem
            ).start()

        def wait_y(buf, sem):
            pltpu.make_async_copy(
                y_hbm.at[pl.ds(0, R * G)], buf, sem
            ).wait()

        start_y(0, buf0, sem0)
        pltpu.sync_copy(t_hbm.at[pl.ds(base, ROWS_W)], tv)

        zero = jnp.zeros((L,), jnp.float32)
        for i in range(T * G):
            accs[i, :] = zero
            accq[i, :] = zero
        accn[...] = zero
        lanes = lax.iota(jnp.int32, L)

        def compute(c, buf):
            t0 = tv[pl.ds(c * R, L)][0]
            t1 = tv[pl.ds(c * R + R - L, L)][L - 1]

            def fast():
                def body(r, acc):
                    s = list(acc[:G])
                    q = list(acc[G:])
                    rg = r * G
                    for g in range(G):
                        yv = buf[rg + g, :]
                        s[g] = s[g] + yv
                        q[g] = q[g] + yv * yv
                    return tuple(s) + tuple(q)

                res = lax.fori_loop(0, R, body, (zero,) * (2 * G), unroll=4)
                tg = t0 * G
                for g in range(G):
                    plsc.addupdate(accs.at[tg + g], res[g])
                    plsc.addupdate(accq.at[tg + g], res[G + g])
                accn[...] = accn[...] + jnp.where(
                    lanes == t0, jnp.float32(R), jnp.float32(0.0)
                )

            def slow():
                def body(j, _):
                    tvec = tv[pl.ds(c * R + j * L, L)]
                    tg0 = tvec[0]
                    tg1 = tvec[L - 1]

                    def grp_uniform():
                        def rbody(r, acc):
                            s = list(acc[:G])
                            q = list(acc[G:])
                            rg = r * G
                            for g in range(G):
                                yv = buf[rg + g, :]
                                s[g] = s[g] + yv
                                q[g] = q[g] + yv * yv
                            return tuple(s) + tuple(q)

                        res = lax.fori_loop(
                            j * L, (j + 1) * L, rbody, (zero,) * (2 * G)
                        )
                        tg = tg0 * G
                        for g in range(G):
                            plsc.addupdate(accs.at[tg + g], res[g])
                            plsc.addupdate(accq.at[tg + g], res[G + g])
                        accn[...] = accn[...] + jnp.where(
                            lanes == tg0, jnp.float32(L), jnp.float32(0.0)
                        )

                    def grp_scatter():
                        cnt = jnp.zeros((L,), jnp.float32)
                        for lane in range(L):
                            t = tvec[lane]
                            tg = t * G
                            rg = (j * L + lane) * G
                            for g in range(G):
                                yv = buf[rg + g, :]
                                plsc.addupdate(accs.at[tg + g], yv)
                                plsc.addupdate(accq.at[tg + g], yv * yv)
                            cnt = cnt + jnp.where(
                                lanes == t, jnp.float32(1.0), jnp.float32(0.0)
                            )
                        accn[...] = accn[...] + cnt

                    lax.cond(tg0 == tg1, grp_uniform, grp_scatter)
                    return 0

                lax.fori_loop(0, R // L, body, 0)

            lax.cond(t0 == t1, fast, slow)

        def pair_body(i, carry):
            c0 = 2 * i
            c1 = c0 + 1

            @pl.when(c1 < NCHUNK)
            def _():
                start_y(c1, buf1, sem1)

            wait_y(buf0, sem0)
            compute(c0, buf0)

            @pl.when(c0 + 2 < NCHUNK)
            def _():
                start_y(c0 + 2, buf0, sem0)

            @pl.when(c1 < NCHUNK)
            def _():
                wait_y(buf1, sem1)
                compute(c1, buf1)

            return carry

        lax.fori_loop(0, (NCHUNK + 1) // 2, pair_body, 0)

        pltpu.make_async_copy(accs, out_s.at[wid], sem0).start()
        pltpu.make_async_copy(accq, out_q.at[wid], sem0).start()
        pltpu.make_async_copy(accn, out_n.at[wid], sem0).start()
        pltpu.make_async_copy(accs, out_s.at[wid], sem0).wait()
        pltpu.make_async_copy(accq, out_q.at[wid], sem0).wait()
        pltpu.make_async_copy(accn, out_n.at[wid], sem0).wait()

    return k(y2d, types)


def _tc_accumulate(y, types_mat):
    nblk = N_TC // BT
    blk0 = N_SC // BT

    def body(y_ref, t_ref, os_ref, oq_ref, on_ref):
        yb = y_ref[...]
        tb = t_ref[0]
        oht = (
            jnp.broadcast_to(tb, (L, BT))
            == lax.broadcasted_iota(jnp.int32, (L, BT), 0)
        ).astype(jnp.float32)
        dn = (((1,), (0,)), ((), ()))
        s = lax.dot_general(oht, yb, dn, preferred_element_type=jnp.float32)
        q = lax.dot_general(oht, yb * yb, dn, preferred_element_type=jnp.float32)
        n = jnp.sum(oht, axis=1, keepdims=True)

        @pl.when(pl.program_id(0) == 0)
        def _():
            os_ref[...] = jnp.zeros_like(os_ref)
            oq_ref[...] = jnp.zeros_like(oq_ref)
            on_ref[...] = jnp.zeros_like(on_ref)

        os_ref[...] += s
        oq_ref[...] += q
        on_ref[...] += n

    return pl.pallas_call(
        body,
        grid=(nblk,),
        in_specs=[
            pl.BlockSpec((BT, D), lambda i: (blk0 + i, 0)),
            pl.BlockSpec((1, 1, BT), lambda i: (blk0 + i, 0, 0)),
        ],
        out_specs=[
            pl.BlockSpec((L, D), lambda i: (0, 0)),
            pl.BlockSpec((L, D), lambda i: (0, 0)),
            pl.BlockSpec((L, 1), lambda i: (0, 0)),
        ],
        out_shape=[
            jax.ShapeDtypeStruct((L, D), jnp.float32),
            jax.ShapeDtypeStruct((L, D), jnp.float32),
            jax.ShapeDtypeStruct((L, 1), jnp.float32),
        ],
    )(y, types_mat)


def _tc_fit(ps, pq, pn, ts, tq, tn):
    def body(ps_ref, pq_ref, pn_ref, ts_ref, tq_ref, tn_ref, out_ref):
        s = ps_ref[...]
        q = pq_ref[...]
        yk = ts_ref[...][:T, :]
        y2k = tq_ref[...][:T, :]
        for w in range(NW):
            yk = yk + s[w * T:(w + 1) * T, :]
            y2k = y2k + q[w * T:(w + 1) * T, :]
        nk16 = jnp.sum(pn_ref[...], axis=0, keepdims=True)
        nkb = jnp.broadcast_to(nk16, (T, L))
        row = lax.broadcasted_iota(jnp.int32, (T, L), 0)
        col = lax.broadcasted_iota(jnp.int32, (T, L), 1)
        nk = (
            jnp.sum(jnp.where(row == col, nkb, 0.0), axis=1, keepdims=True)
            + tn_ref[...][:T, :]
        )

        y_norm = jnp.sqrt(jnp.sum(yk * yk, axis=1, keepdims=True))
        y2_norm = jnp.sqrt(jnp.sum(y2k * y2k, axis=1, keepdims=True))
        nsafe = jnp.maximum(nk, 1.0)
        var = y2_norm / nsafe - (y_norm / nsafe) ** 2
        sc = jnp.sqrt(jnp.maximum(var, 1e-20))
        sc = jnp.where(nk > 0, sc, jnp.ones_like(sc))
        out_ref[...] = jnp.broadcast_to(sc, (T, D))

    out = pl.pallas_call(
        body,
        out_shape=jax.ShapeDtypeStruct((T, D), jnp.float32),
    )(ps, pq, pn, ts, tq, tn)
    return out[:, :1]


def kernel(Y, types):
    yflat = Y.reshape(N_ROWS * G, L)
    ps, pq, pn = _sc_accumulate(yflat, types)
    ts, tq, tn = _tc_accumulate(Y, types.reshape(N_ROWS // BT, 1, BT))
    return _tc_fit(ps.reshape(NW * T, D), pq.reshape(NW * T, D), pn, ts, tq, tn)
